# baseline (speedup 1.0000x reference)
import functools

import jax
import jax.numpy as jnp
from jax import lax
from jax.experimental import pallas as pl
from jax.experimental.pallas import tpu as pltpu
from jax.experimental.pallas import tpu_sc as plsc

_VOCAB = 1000000
_D = 64
_B = 4096
_T = 200
_N = _B * _T
_NC = 2
_NS = 16
_NW = _NC * _NS
_PER_W = _N // _NW
_CH = 320
_NCH = _PER_W // _CH
_NB = 4
_SCALE = 8.0

_mesh = plsc.VectorSubcoreMesh(core_axis_name="c", subcore_axis_name="s")


@functools.partial(
    pl.kernel,
    mesh=_mesh,
    out_type=jax.ShapeDtypeStruct((_N, 2 * _D), jnp.float32),
    scratch_types=(
        [pltpu.VMEM((_PER_W,), jnp.int32)]
        + [pltpu.VMEM((_CH, _D), jnp.float32)] * _NB
        + [pltpu.SemaphoreType.DMA] * (2 * _NB)
    ),
    compiler_params=pltpu.CompilerParams(
        use_tc_tiling_on_sc=False, needs_layout_passes=False
    ),
)
def _embed(idx_hbm, table_hbm, out_hbm, idx_v, *scratch):
    bufs = scratch[:_NB]
    gsems = scratch[_NB:2 * _NB]
    ssems = scratch[2 * _NB:]

    wid = lax.axis_index("s") * _NC + lax.axis_index("c")
    base = wid * _PER_W
    pltpu.sync_copy(idx_hbm.at[pl.ds(base, _PER_W)], idx_v)

    def gather_desc(c, b):
        src = table_hbm.at[idx_v.at[pl.ds(c * _CH, _CH)]]
        return pltpu.make_async_copy(src, bufs[b], gsems[b])

    def store_desc(c, b):
        dst = out_hbm.at[pl.ds(base + c * _CH, _CH), pl.ds(0, _D)]
        return pltpu.make_async_copy(bufs[b], dst, ssems[b])

    def head(c, b):
        gather_desc(c, b).wait()
        store_desc(c, b).start()

    def tail(c, b):
        nb = (b + 2) % _NB
        store_desc(c - 2, nb).wait()
        gather_desc(c + 2, nb).start()

    gather_desc(0, 0).start()
    gather_desc(1, 1).start()
    head(0, 0)
    gather_desc(2, 2).start()
    head(1, 1)
    gather_desc(3, 3).start()
    head(2, 2)
    tail(2, 2)
    head(3, 3)
    tail(3, 3)

    def step(o, carry):
        c0 = o * _NB
        for u in range(_NB):
            head(c0 + u, u)
            tail(c0 + u, u)
        return carry

    lax.fori_loop(1, _NCH // _NB - 1, step, 0, unroll=False)

    head(_NCH - 4, 0)
    tail(_NCH - 4, 0)
    head(_NCH - 3, 1)
    tail(_NCH - 3, 1)
    head(_NCH - 2, 2)
    head(_NCH - 1, 3)
    store_desc(_NCH - 4, 0).wait()
    store_desc(_NCH - 3, 1).wait()
    store_desc(_NCH - 2, 2).wait()
    store_desc(_NCH - 1, 3).wait()


_RCH = 400
_NRCH = _VOCAB // _RCH


@functools.partial(
    pl.kernel,
    mesh=_mesh,
    out_type=jax.ShapeDtypeStruct((_VOCAB * _D,), jnp.float32),
    scratch_types=[
        pltpu.VMEM((_RCH, _D), jnp.float32),
        pltpu.VMEM((_RCH * _D,), jnp.float32),
    ],
    compiler_params=pltpu.CompilerParams(
        use_tc_tiling_on_sc=True, needs_layout_passes=False
    ),
)
def _prep(table_hbm, out_hbm, buf_a, buf_b):
    w = lax.axis_index("s") * _NC + lax.axis_index("c")

    def scale_copy(r, carry):
        for k in range(_D // 16):
            buf_b[pl.ds(r * _D + 16 * k, 16)] = (
                buf_a[r, pl.ds(16 * k, 16)] * _SCALE
            )
        return carry

    def step(k, carry):
        c = w + _NW * k

        @pl.when(c < _NRCH)
        def _():
            pltpu.sync_copy(table_hbm.at[pl.ds(c * _RCH, _RCH), :], buf_a)
            lax.fori_loop(0, _RCH, scale_copy, 0, unroll=8)
            pltpu.sync_copy(buf_b, out_hbm.at[pl.ds(c * _RCH * _D, _RCH * _D)])

        return carry

    lax.fori_loop(0, (_NRCH + _NW - 1) // _NW, step, 0, unroll=False)


def kernel(x, table):
    idx = x.reshape(_N)
    tlin = _prep(table).reshape(_VOCAB, _D)
    outp = _embed(idx, tlin)
    return outp[:, :_D].reshape(_B, _T, _D)


# speedup vs baseline: 1.2391x; 1.2391x over previous
import functools

import jax
import jax.numpy as jnp
from jax import lax
from jax.experimental import pallas as pl
from jax.experimental.pallas import tpu as pltpu
from jax.experimental.pallas import tpu_sc as plsc

_VOCAB = 1000000
_D = 64
_B = 4096
_T = 200
_N = _B * _T
_NC = 2
_NS = 16
_NW = _NC * _NS
_PER_W = _N // _NW
_CH = 320
_NCH = _PER_W // _CH
_NB = 4
_SCALE = 8.0

_mesh = plsc.VectorSubcoreMesh(core_axis_name="c", subcore_axis_name="s")


@functools.partial(
    pl.kernel,
    mesh=_mesh,
    out_type=jax.ShapeDtypeStruct((_N, 2 * _D), jnp.float32),
    scratch_types=(
        [pltpu.VMEM((_PER_W,), jnp.int32)]
        + [pltpu.VMEM((_CH, _D), jnp.float32)] * _NB
        + [pltpu.SemaphoreType.DMA] * (2 * _NB)
    ),
    compiler_params=pltpu.CompilerParams(
        use_tc_tiling_on_sc=False, needs_layout_passes=False
    ),
)
def _embed(idx_hbm, table_hbm, out_hbm, idx_v, *scratch):
    bufs = scratch[:_NB]
    gsems = scratch[_NB:2 * _NB]
    ssems = scratch[2 * _NB:]

    wid = lax.axis_index("s") * _NC + lax.axis_index("c")
    base = wid * _PER_W
    pltpu.sync_copy(idx_hbm.at[pl.ds(base, _PER_W)], idx_v)

    def gather_desc(c, b):
        src = table_hbm.at[idx_v.at[pl.ds(c * _CH, _CH)]]
        return pltpu.make_async_copy(src, bufs[b], gsems[b])

    def store_desc(c, b):
        dst = out_hbm.at[pl.ds(base + c * _CH, _CH), pl.ds(0, _D)]
        return pltpu.make_async_copy(bufs[b], dst, ssems[b])

    def head(c, b):
        gather_desc(c, b).wait()
        store_desc(c, b).start()

    def tail(c, b):
        nb = (b + 2) % _NB
        store_desc(c - 2, nb).wait()
        gather_desc(c + 2, nb).start()

    gather_desc(0, 0).start()
    gather_desc(1, 1).start()
    head(0, 0)
    gather_desc(2, 2).start()
    head(1, 1)
    gather_desc(3, 3).start()
    head(2, 2)
    tail(2, 2)
    head(3, 3)
    tail(3, 3)

    def step(o, carry):
        c0 = o * _NB
        for u in range(_NB):
            head(c0 + u, u)
            tail(c0 + u, u)
        return carry

    lax.fori_loop(1, _NCH // _NB - 1, step, 0, unroll=False)

    head(_NCH - 4, 0)
    tail(_NCH - 4, 0)
    head(_NCH - 3, 1)
    tail(_NCH - 3, 1)
    head(_NCH - 2, 2)
    head(_NCH - 1, 3)
    store_desc(_NCH - 4, 0).wait()
    store_desc(_NCH - 3, 1).wait()
    store_desc(_NCH - 2, 2).wait()
    store_desc(_NCH - 1, 3).wait()


_RCH = 200
_NRCH = _VOCAB // _RCH


@functools.partial(
    pl.kernel,
    mesh=_mesh,
    out_type=jax.ShapeDtypeStruct((_VOCAB * _D,), jnp.float32),
    scratch_types=(
        [pltpu.VMEM((_RCH, _D), jnp.float32)] * 2
        + [pltpu.VMEM((_RCH * _D,), jnp.float32)] * 2
        + [pltpu.SemaphoreType.DMA] * 4
    ),
    compiler_params=pltpu.CompilerParams(
        use_tc_tiling_on_sc=True, needs_layout_passes=False
    ),
)
def _prep(table_hbm, out_hbm, ba0, ba1, bb0, bb1, r0, r1, w0, w1):
    bas, bbs, rsems, wsems = (ba0, ba1), (bb0, bb1), (r0, r1), (w0, w1)
    w = lax.axis_index("s") * _NC + lax.axis_index("c")
    nfull = _NRCH // _NW
    tail_c = nfull * _NW + w

    def read_desc(c, p):
        return pltpu.make_async_copy(
            table_hbm.at[pl.ds(c * _RCH, _RCH), :], bas[p], rsems[p]
        )

    def write_desc(c, p):
        dst = out_hbm.at[pl.ds(c * _RCH * _D, _RCH * _D)]
        return pltpu.make_async_copy(bbs[p], dst, wsems[p])

    def scale_copy(p):
        ba, bb = bas[p], bbs[p]

        def row(r, carry):
            for k in range(_D // 16):
                bb[pl.ds(r * _D + 16 * k, 16)] = ba[r, pl.ds(16 * k, 16)] * _SCALE
            return carry

        lax.fori_loop(0, _RCH, row, 0, unroll=8)

    def cix(j):
        return w + _NW * j

    read_desc(cix(0), 0).start()
    read_desc(cix(1), 1).start()
    read_desc(cix(0), 0).wait()
    scale_copy(0)
    read_desc(cix(2), 0).start()
    write_desc(cix(0), 0).start()
    read_desc(cix(1), 1).wait()
    scale_copy(1)
    read_desc(cix(3), 1).start()
    write_desc(cix(1), 1).start()

    def step(i, carry):
        for p in range(2):
            j = 2 * i + p
            c = cix(j)
            read_desc(c, p).wait()
            write_desc(cix(j - 2), p).wait()
            scale_copy(p)
            read_desc(cix(j + 2), p).start()
            write_desc(c, p).start()
        return carry

    lax.fori_loop(1, nfull // 2 - 1, step, 0, unroll=False)

    read_desc(cix(nfull - 2), 0).wait()
    write_desc(cix(nfull - 4), 0).wait()
    scale_copy(0)
    write_desc(cix(nfull - 2), 0).start()

    @pl.when(w < _NRCH - nfull * _NW)
    def _():
        pltpu.make_async_copy(
            table_hbm.at[pl.ds(tail_c * _RCH, _RCH), :], bas[0], rsems[0]
        ).start()

    read_desc(cix(nfull - 1), 1).wait()
    write_desc(cix(nfull - 3), 1).wait()
    scale_copy(1)
    write_desc(cix(nfull - 1), 1).start()
    write_desc(cix(nfull - 2), 0).wait()

    @pl.when(w < _NRCH - nfull * _NW)
    def _():
        pltpu.make_async_copy(
            table_hbm.at[pl.ds(tail_c * _RCH, _RCH), :], bas[0], rsems[0]
        ).wait()
        scale_copy(0)
        write_desc(tail_c, 0).start()
        write_desc(tail_c, 0).wait()

    write_desc(cix(nfull - 1), 1).wait()


def kernel(x, table):
    idx = x.reshape(_N)
    tlin = _prep(table).reshape(_VOCAB, _D)
    outp = _embed(idx, tlin)
    return outp[:, :_D].reshape(_B, _T, _D)


# speedup vs baseline: 1.6617x; 1.3411x over previous
import functools

import jax
import jax.numpy as jnp
from jax import lax
from jax.experimental import pallas as pl
from jax.experimental.pallas import tpu as pltpu
from jax.experimental.pallas import tpu_sc as plsc

_VOCAB = 1000000
_D = 64
_B = 4096
_T = 200
_N = _B * _T
_NC = 2
_NS = 16
_NW = _NC * _NS
_PER_W = _N // _NW
_CH = 320
_NCH = _PER_W // _CH
_NB = 4
_SCALE = 8.0

_mesh = plsc.VectorSubcoreMesh(core_axis_name="c", subcore_axis_name="s")


@functools.partial(
    pl.kernel,
    mesh=_mesh,
    out_type=jax.ShapeDtypeStruct((_N, 2 * _D), jnp.float32),
    scratch_types=(
        [pltpu.VMEM((_PER_W,), jnp.int32)]
        + [pltpu.VMEM((_CH, _D), jnp.float32)] * _NB
        + [pltpu.SemaphoreType.DMA] * (2 * _NB)
    ),
    compiler_params=pltpu.CompilerParams(
        use_tc_tiling_on_sc=False, needs_layout_passes=False
    ),
)
def _embed(idx_hbm, table_hbm, out_hbm, idx_v, *scratch):
    bufs = scratch[:_NB]
    gsems = scratch[_NB:2 * _NB]
    ssems = scratch[2 * _NB:]

    wid = lax.axis_index("s") * _NC + lax.axis_index("c")
    base = wid * _PER_W
    pltpu.sync_copy(idx_hbm.at[pl.ds(base, _PER_W)], idx_v)

    def gather_desc(c, b):
        src = table_hbm.at[idx_v.at[pl.ds(c * _CH, _CH)]]
        return pltpu.make_async_copy(src, bufs[b], gsems[b])

    def store_desc(c, b):
        dst = out_hbm.at[pl.ds(base + c * _CH, _CH), pl.ds(0, _D)]
        return pltpu.make_async_copy(bufs[b], dst, ssems[b])

    def scale(b):
        buf = bufs[b]

        def row(r, carry):
            for k in range(_D // 16):
                sl = (r, pl.ds(16 * k, 16))
                buf[sl] = buf[sl] * _SCALE
            return carry

        lax.fori_loop(0, _CH, row, 0, unroll=8)

    def head(c, b):
        gather_desc(c, b).wait()
        scale(b)
        store_desc(c, b).start()

    def tail(c, b):
        nb = (b + 2) % _NB
        store_desc(c - 2, nb).wait()
        gather_desc(c + 2, nb).start()

    gather_desc(0, 0).start()
    gather_desc(1, 1).start()
    head(0, 0)
    gather_desc(2, 2).start()
    head(1, 1)
    gather_desc(3, 3).start()
    head(2, 2)
    tail(2, 2)
    head(3, 3)
    tail(3, 3)

    def step(o, carry):
        c0 = o * _NB
        for u in range(_NB):
            head(c0 + u, u)
            tail(c0 + u, u)
        return carry

    lax.fori_loop(1, _NCH // _NB - 1, step, 0, unroll=False)

    head(_NCH - 4, 0)
    tail(_NCH - 4, 0)
    head(_NCH - 3, 1)
    tail(_NCH - 3, 1)
    head(_NCH - 2, 2)
    head(_NCH - 1, 3)
    store_desc(_NCH - 4, 0).wait()
    store_desc(_NCH - 3, 1).wait()
    store_desc(_NCH - 2, 2).wait()
    store_desc(_NCH - 1, 3).wait()


def kernel(x, table):
    idx = x.reshape(_N)
    outp = _embed(idx, table)
    return outp[:, :_D].reshape(_B, _T, _D)
